# baseline (speedup 1.0000x reference)
import functools

import jax
import jax.numpy as jnp
from jax import lax
from jax.experimental import pallas as pl
from jax.experimental.pallas import tpu as pltpu
from jax.experimental.pallas import tpu_sc as plsc

D, F, E, K, NSH = 2048, 5120, 8, 2, 2
T = 2048
BM = 256
BF = 512
NF = F // BF
NB = 23
P = NB * BM
BM2 = 512

_NW = 32
_GCH = 8
_CCH = 16

_MESH = plsc.VectorSubcoreMesh(core_axis_name="c", subcore_axis_name="s", num_cores=2, num_subcores=16)


def _sc_dispatch_gather(x, eflat, wflat):
    NCH = (T * K) // 16
    bpw = P // _NW
    nit = bpw // _GCH

    @functools.partial(
        pl.kernel, mesh=_MESH,
        compiler_params=pltpu.CompilerParams(
            needs_layout_passes=False, use_tc_tiling_on_sc=True),
        out_type=(
            jax.ShapeDtypeStruct((P, D), jnp.float32),
            jax.ShapeDtypeStruct((P,), jnp.float32),
            jax.ShapeDtypeStruct((T,), jnp.int32),
            jax.ShapeDtypeStruct((T,), jnp.int32),
            jax.ShapeDtypeStruct((32,), jnp.int32),
        ),
        scratch_types=[
            pltpu.VMEM((T * K,), jnp.int32),
            pltpu.VMEM((T * K,), jnp.float32),
            pltpu.VMEM((P,), jnp.int32),
            pltpu.VMEM((P,), jnp.float32),
            pltpu.VMEM((T,), jnp.int32),
            pltpu.VMEM((T,), jnp.int32),
            pltpu.VMEM((16,), jnp.int32),
            pltpu.VMEM((32,), jnp.int32),
            pltpu.VMEM((2, _GCH, D), jnp.float32),
            pltpu.SemaphoreType.DMA,
            pltpu.SemaphoreType.DMA,
        ],
    )
    def k(x_hbm, e_hbm, w_hbm, xg_hbm, ws_hbm, p0_hbm, p1_hbm, be_hbm,
          e_v, w_v, perm_v, ws_v, pos0_v, pos1_v, base_ref, be_v, rows_v,
          sem0, sem1):
        wid = lax.axis_index("s") * 2 + lax.axis_index("c")
        base = wid * bpw
        pltpu.sync_copy(e_hbm, e_v)
        pltpu.sync_copy(w_hbm, w_v)
        lanes = lax.iota(jnp.int32, 16)

        def initp(g, carry):
            perm_v[pl.ds(g * 16, 16)] = (g * 16 + lanes) & (T - 1)
            ws_v[pl.ds(g * 16, 16)] = jnp.zeros((16,), jnp.float32)
            return carry

        lax.fori_loop(0, P // 16, initp, 0)

        def hist_body(g, hist):
            ev = e_v[pl.ds(g * 16, 16)]
            for e in range(E):
                c = plsc.all_reduce_population_count(ev == e)
                hist = hist + jnp.where(lanes == e, c, 0)
            return hist

        hist = lax.fori_loop(0, NCH, hist_body, jnp.zeros((16,), jnp.int32))
        padded = ((hist + (BM - 1)) >> 8) << 8
        cum = plsc.cumsum(padded)
        poff = cum - padded
        base_ref[...] = poff

        @pl.when(wid == 0)
        def _():
            for ch in range(2):
                bstart = (ch * 16 + lanes) * BM
                acc = jnp.zeros((16,), jnp.int32)
                for e in range(E):
                    acc = acc + jnp.where(bstart >= cum[e], 1, 0)
                be_v[pl.ds(ch * 16, 16)] = acc
            pltpu.sync_copy(be_v, be_hbm)

        def body(g, carry):
            ev = e_v[pl.ds(g * 16, 16)]
            wv = w_v[pl.ds(g * 16, 16)]
            basev = plsc.load_gather(base_ref, [ev])
            rank = jnp.zeros((16,), jnp.int32)
            newbase = base_ref[...]
            for e in range(E):
                m = ev == e
                cs = plsc.cumsum(jnp.where(m, 1, 0))
                rank = jnp.where(m, cs - 1, rank)
                cnt = plsc.all_reduce_population_count(m)
                newbase = newbase + jnp.where(lanes == e, cnt, 0)
            base_ref[...] = newbase
            dest = basev + rank
            tokv = (g * 16 + lanes) >> 1
            plsc.store_scatter(perm_v, [dest], tokv)
            plsc.store_scatter(ws_v, [dest], wv)
            even = (lanes & 1) == 0
            plsc.store_scatter(pos0_v, [tokv], dest, mask=even)
            plsc.store_scatter(pos1_v, [tokv], dest, mask=jnp.logical_not(even))
            return carry

        lax.fori_loop(0, NCH, body, 0)

        @pl.when(wid == 0)
        def _():
            pltpu.sync_copy(ws_v, ws_hbm)
            pltpu.sync_copy(pos0_v, p0_hbm)
            pltpu.sync_copy(pos1_v, p1_hbm)

        def idx(g):
            return perm_v.at[pl.ds(base + g * _GCH, _GCH)]

        pltpu.async_copy(x_hbm.at[idx(0)], rows_v.at[0], sem0)

        def gbody(h, carry):
            g0 = 2 * h

            @pl.when(g0 + 1 < nit)
            def _():
                pltpu.async_copy(x_hbm.at[idx(g0 + 1)], rows_v.at[1], sem1)

            pltpu.make_async_copy(x_hbm.at[idx(0)], rows_v.at[0], sem0).wait()
            pltpu.sync_copy(rows_v.at[0], xg_hbm.at[pl.ds(base + g0 * _GCH, _GCH)])

            @pl.when(g0 + 2 < nit)
            def _():
                pltpu.async_copy(x_hbm.at[idx(g0 + 2)], rows_v.at[0], sem0)

            @pl.when(g0 + 1 < nit)
            def _():
                pltpu.make_async_copy(x_hbm.at[idx(0)], rows_v.at[1], sem1).wait()
                pltpu.sync_copy(rows_v.at[1],
                                xg_hbm.at[pl.ds(base + (g0 + 1) * _GCH, _GCH)])

            return carry

        lax.fori_loop(0, (nit + 1) // 2, gbody, 0)

    return k(x, eflat, wflat)


def _sc_combine(out_sh, yg, pos0, pos1):
    tpw = T // _NW
    nit = tpw // _CCH

    @functools.partial(
        pl.kernel, mesh=_MESH,
        compiler_params=pltpu.CompilerParams(use_tc_tiling_on_sc=True),
        out_type=jax.ShapeDtypeStruct((T, D), jnp.float32),
        scratch_types=[
            pltpu.VMEM((nit, _CCH), jnp.int32),
            pltpu.VMEM((nit, _CCH), jnp.int32),
            pltpu.VMEM((_CCH, D), jnp.float32),
            pltpu.VMEM((_CCH, D), jnp.float32),
            pltpu.VMEM((_CCH, D), jnp.float32),
            pltpu.SemaphoreType.DMA,
            pltpu.SemaphoreType.DMA,
        ],
    )
    def k(sh_hbm, yg_hbm, p0_hbm, p1_hbm, out_hbm, p0_v, p1_v,
          a_v, b_v, c_v, sem0, sem1):
        wid = lax.axis_index("s") * 2 + lax.axis_index("c")
        base = wid * tpw
        pltpu.sync_copy(p0_hbm.at[wid], p0_v)
        pltpu.sync_copy(p1_hbm.at[wid], p1_v)

        def body(g, carry):
            cpa = pltpu.async_copy(yg_hbm.at[p0_v.at[g]], a_v, sem0)
            cpb = pltpu.async_copy(yg_hbm.at[p1_v.at[g]], b_v, sem1)
            pltpu.sync_copy(sh_hbm.at[pl.ds(base + g * _CCH, _CCH)], c_v)
            cpa.wait()
            cpb.wait()

            def row(r, carry2):
                def col(kk, carry3):
                    for q in range(4):
                        sl = pl.ds(kk * 64 + q * 16, 16)
                        plsc.addupdate(c_v.at[r, sl], a_v[r, sl] + b_v[r, sl])
                    return carry3
                lax.fori_loop(0, D // 64, col, 0)
                return carry2

            lax.fori_loop(0, _CCH, row, 0)
            pltpu.sync_copy(c_v, out_hbm.at[pl.ds(base + g * _CCH, _CCH)])
            return carry

        lax.fori_loop(0, nit, body, 0)

    return k(out_sh, yg, pos0.reshape(_NW, nit, _CCH), pos1.reshape(_NW, nit, _CCH))


def _routed_body(be_ref, x_ref, wg_ref, wu_ref, wd_ref, ws_ref, out_ref):
    b = pl.program_id(0)
    f = pl.program_id(1)

    @pl.when(f == 0)
    def _():
        out_ref[...] = jnp.zeros_like(out_ref)

    @pl.when(be_ref[b] < E)
    def _():
        x = x_ref[...]
        g = jnp.dot(x, wg_ref[0], preferred_element_type=jnp.float32)
        u = jnp.dot(x, wu_ref[0], preferred_element_type=jnp.float32)
        h = (g * jax.nn.sigmoid(g)) * u
        out_ref[...] += jnp.dot(h, wd_ref[0], preferred_element_type=jnp.float32)

    @pl.when(f == NF - 1)
    def _():
        out_ref[...] *= ws_ref[0, 0, :][:, None]


def _routed_mlp(block_expert, xg, Wg, Wu, Wd, wslot):
    grid_spec = pltpu.PrefetchScalarGridSpec(
        num_scalar_prefetch=1,
        grid=(NB, NF),
        in_specs=[
            pl.BlockSpec((BM, D), lambda b, f, be: (b, 0)),
            pl.BlockSpec((1, D, BF), lambda b, f, be: (jnp.minimum(be[b], E - 1), 0, f)),
            pl.BlockSpec((1, D, BF), lambda b, f, be: (jnp.minimum(be[b], E - 1), 0, f)),
            pl.BlockSpec((1, BF, D), lambda b, f, be: (jnp.minimum(be[b], E - 1), f, 0)),
            pl.BlockSpec((1, 1, BM), lambda b, f, be: (b, 0, 0)),
        ],
        out_specs=pl.BlockSpec((BM, D), lambda b, f, be: (b, 0)),
    )
    return pl.pallas_call(
        _routed_body,
        grid_spec=grid_spec,
        out_shape=jax.ShapeDtypeStruct((P, D), jnp.float32),
    )(block_expert, xg, Wg, Wu, Wd, wslot.reshape(NB, 1, BM))


def _shared_body(x_ref, wg_ref, wu_ref, wd_ref, out_ref):
    si = pl.program_id(1)
    f = pl.program_id(2)

    @pl.when((si == 0) & (f == 0))
    def _():
        out_ref[...] = jnp.zeros_like(out_ref)

    x = x_ref[...]
    g = jnp.dot(x, wg_ref[0], preferred_element_type=jnp.float32)
    u = jnp.dot(x, wu_ref[0], preferred_element_type=jnp.float32)
    h = (g * jax.nn.sigmoid(g)) * u
    out_ref[...] += jnp.dot(h, wd_ref[0], preferred_element_type=jnp.float32)


def _shared_mlp(x, Wg_s, Wu_s, Wd_s):
    return pl.pallas_call(
        _shared_body,
        grid=(T // BM2, NSH, NF),
        in_specs=[
            pl.BlockSpec((BM2, D), lambda tb, si, f: (tb, 0)),
            pl.BlockSpec((1, D, BF), lambda tb, si, f: (si, 0, f)),
            pl.BlockSpec((1, D, BF), lambda tb, si, f: (si, 0, f)),
            pl.BlockSpec((1, BF, D), lambda tb, si, f: (si, f, 0)),
        ],
        out_specs=pl.BlockSpec((BM2, D), lambda tb, si, f: (tb, 0)),
        out_shape=jax.ShapeDtypeStruct((T, D), jnp.float32),
    )(x, Wg_s, Wu_s, Wd_s)


def kernel(hidden_states, gate_w, Wg, Wu, Wd, Wg_s, Wu_s, Wd_s):
    b, s, d = hidden_states.shape
    x = hidden_states.reshape(-1, d)

    logits = x @ gate_w.T
    scores = jax.nn.sigmoid(logits)
    topk_w, topk_idx = jax.lax.top_k(scores, K)
    topk_w = topk_w / jnp.sum(topk_w, axis=-1, keepdims=True)

    xg, wslot, pos0, pos1, block_expert = _sc_dispatch_gather(
        x, topk_idx.reshape(-1).astype(jnp.int32), topk_w.reshape(-1))

    yg = _routed_mlp(block_expert, xg, Wg, Wu, Wd, wslot)
    out_sh = _shared_mlp(x, Wg_s, Wu_s, Wd_s)
    out = _sc_combine(out_sh, yg, pos0, pos1)
    return out.reshape(b, s, d)


# speedup vs baseline: 1.0052x; 1.0052x over previous
import functools

import jax
import jax.numpy as jnp
from jax import lax
from jax.experimental import pallas as pl
from jax.experimental.pallas import tpu as pltpu
from jax.experimental.pallas import tpu_sc as plsc

D, F, E, K, NSH = 2048, 5120, 8, 2, 2
T = 2048
BM = 256
BF = 512
NF = F // BF
NB = 23
P = NB * BM
BM2 = 512

_NW = 32
_GCH = 8
_CCH = 16

def _mesh():
    return plsc.VectorSubcoreMesh(core_axis_name="c", subcore_axis_name="s",
                                  num_cores=2, num_subcores=16)


RBM = 256


def _router_body(x_ref, gw_ref, e0_ref, e1_ref, w0_ref, w1_ref):
    l = jnp.dot(x_ref[...], gw_ref[...], preferred_element_type=jnp.float32)
    sc = jax.nn.sigmoid(l)
    s1 = sc[:, 0]
    i1 = jnp.zeros((RBM,), jnp.int32)
    for e in range(1, E):
        upd = sc[:, e] > s1
        s1 = jnp.where(upd, sc[:, e], s1)
        i1 = jnp.where(upd, e, i1)
    s2 = jnp.full((RBM,), -jnp.inf, jnp.float32)
    i2 = jnp.zeros((RBM,), jnp.int32)
    for e in range(E):
        cand = jnp.logical_and(i1 != e, sc[:, e] > s2)
        s2 = jnp.where(cand, sc[:, e], s2)
        i2 = jnp.where(cand, e, i2)
    tot = s1 + s2
    e0_ref[0, 0, :] = i1
    e1_ref[0, 0, :] = i2
    w0_ref[0, 0, :] = s1 / tot
    w1_ref[0, 0, :] = s2 / tot


def _router(x, gate_w):
    gwp = jnp.zeros((D, 128), jnp.float32).at[:, :E].set(gate_w.T)
    nblk = T // RBM
    outs = pl.pallas_call(
        _router_body,
        grid=(nblk,),
        in_specs=[
            pl.BlockSpec((RBM, D), lambda tb: (tb, 0)),
            pl.BlockSpec((D, 128), lambda tb: (0, 0)),
        ],
        out_specs=[
            pl.BlockSpec((1, 1, RBM), lambda tb: (tb, 0, 0)),
            pl.BlockSpec((1, 1, RBM), lambda tb: (tb, 0, 0)),
            pl.BlockSpec((1, 1, RBM), lambda tb: (tb, 0, 0)),
            pl.BlockSpec((1, 1, RBM), lambda tb: (tb, 0, 0)),
        ],
        out_shape=[
            jax.ShapeDtypeStruct((nblk, 1, RBM), jnp.int32),
            jax.ShapeDtypeStruct((nblk, 1, RBM), jnp.int32),
            jax.ShapeDtypeStruct((nblk, 1, RBM), jnp.float32),
            jax.ShapeDtypeStruct((nblk, 1, RBM), jnp.float32),
        ],
    )(x, gwp)
    return outs


def _sc_dispatch_gather(x, eflat, wflat):
    NCH = (T * K) // 16
    bpw = P // _NW
    nit = bpw // _GCH

    @functools.partial(
        pl.kernel, mesh=_mesh(),
        compiler_params=pltpu.CompilerParams(
            needs_layout_passes=False, use_tc_tiling_on_sc=True),
        out_type=(
            jax.ShapeDtypeStruct((P, D), jnp.float32),
            jax.ShapeDtypeStruct((P,), jnp.float32),
            jax.ShapeDtypeStruct((T,), jnp.int32),
            jax.ShapeDtypeStruct((T,), jnp.int32),
            jax.ShapeDtypeStruct((32,), jnp.int32),
        ),
        scratch_types=[
            pltpu.VMEM((T * K,), jnp.int32),
            pltpu.VMEM((T * K,), jnp.float32),
            pltpu.VMEM((P,), jnp.int32),
            pltpu.VMEM((P,), jnp.float32),
            pltpu.VMEM((T,), jnp.int32),
            pltpu.VMEM((T,), jnp.int32),
            pltpu.VMEM((16,), jnp.int32),
            pltpu.VMEM((32,), jnp.int32),
            pltpu.VMEM((2, _GCH, D), jnp.float32),
            pltpu.SemaphoreType.DMA,
            pltpu.SemaphoreType.DMA,
        ],
    )
    def k(x_hbm, e_hbm, w_hbm, xg_hbm, ws_hbm, p0_hbm, p1_hbm, be_hbm,
          e_v, w_v, perm_v, ws_v, pos0_v, pos1_v, base_ref, be_v, rows_v,
          sem0, sem1):
        wid = lax.axis_index("s") * 2 + lax.axis_index("c")
        base = wid * bpw
        pltpu.sync_copy(e_hbm, e_v)
        pltpu.sync_copy(w_hbm, w_v)
        lanes = lax.iota(jnp.int32, 16)

        def initp(g, carry):
            perm_v[pl.ds(g * 16, 16)] = (g * 16 + lanes) & (T - 1)
            ws_v[pl.ds(g * 16, 16)] = jnp.zeros((16,), jnp.float32)
            return carry

        lax.fori_loop(0, P // 16, initp, 0)

        def hist_body(g, hist):
            ev = e_v[pl.ds(g * 16, 16)]
            for e in range(E):
                c = plsc.all_reduce_population_count(ev == e)
                hist = hist + jnp.where(lanes == e, c, 0)
            return hist

        hist = lax.fori_loop(0, NCH, hist_body, jnp.zeros((16,), jnp.int32))
        padded = ((hist + (BM - 1)) >> 8) << 8
        cum = plsc.cumsum(padded)
        poff = cum - padded
        base_ref[...] = poff

        @pl.when(wid == 0)
        def _():
            for ch in range(2):
                bstart = (ch * 16 + lanes) * BM
                acc = jnp.zeros((16,), jnp.int32)
                for e in range(E):
                    acc = acc + jnp.where(bstart >= cum[e], 1, 0)
                be_v[pl.ds(ch * 16, 16)] = acc
            pltpu.sync_copy(be_v, be_hbm)

        def body(g, carry):
            ev = e_v[pl.ds(g * 16, 16)]
            wv = w_v[pl.ds(g * 16, 16)]
            basev = plsc.load_gather(base_ref, [ev])
            rank = jnp.zeros((16,), jnp.int32)
            newbase = base_ref[...]
            for e in range(E):
                m = ev == e
                cs = plsc.cumsum(jnp.where(m, 1, 0))
                rank = jnp.where(m, cs - 1, rank)
                cnt = plsc.all_reduce_population_count(m)
                newbase = newbase + jnp.where(lanes == e, cnt, 0)
            base_ref[...] = newbase
            dest = basev + rank
            tokv = (g * 16 + lanes) >> 1
            plsc.store_scatter(perm_v, [dest], tokv)
            plsc.store_scatter(ws_v, [dest], wv)
            even = (lanes & 1) == 0
            plsc.store_scatter(pos0_v, [tokv], dest, mask=even)
            plsc.store_scatter(pos1_v, [tokv], dest, mask=jnp.logical_not(even))
            return carry

        lax.fori_loop(0, NCH, body, 0)

        @pl.when(wid == 0)
        def _():
            pltpu.sync_copy(ws_v, ws_hbm)
            pltpu.sync_copy(pos0_v, p0_hbm)
            pltpu.sync_copy(pos1_v, p1_hbm)

        def idx(g):
            return perm_v.at[pl.ds(base + g * _GCH, _GCH)]

        pltpu.async_copy(x_hbm.at[idx(0)], rows_v.at[0], sem0)

        def gbody(h, carry):
            g0 = 2 * h

            @pl.when(g0 + 1 < nit)
            def _():
                pltpu.async_copy(x_hbm.at[idx(g0 + 1)], rows_v.at[1], sem1)

            pltpu.make_async_copy(x_hbm.at[idx(0)], rows_v.at[0], sem0).wait()
            pltpu.sync_copy(rows_v.at[0], xg_hbm.at[pl.ds(base + g0 * _GCH, _GCH)])

            @pl.when(g0 + 2 < nit)
            def _():
                pltpu.async_copy(x_hbm.at[idx(g0 + 2)], rows_v.at[0], sem0)

            @pl.when(g0 + 1 < nit)
            def _():
                pltpu.make_async_copy(x_hbm.at[idx(0)], rows_v.at[1], sem1).wait()
                pltpu.sync_copy(rows_v.at[1],
                                xg_hbm.at[pl.ds(base + (g0 + 1) * _GCH, _GCH)])

            return carry

        lax.fori_loop(0, (nit + 1) // 2, gbody, 0)

    return k(x, eflat, wflat)


def _sc_combine(out_sh, yg, pos0, pos1):
    tpw = T // _NW
    nit = tpw // _CCH

    @functools.partial(
        pl.kernel, mesh=_mesh(),
        compiler_params=pltpu.CompilerParams(use_tc_tiling_on_sc=True),
        out_type=jax.ShapeDtypeStruct((T, D), jnp.float32),
        scratch_types=[
            pltpu.VMEM((nit, _CCH), jnp.int32),
            pltpu.VMEM((nit, _CCH), jnp.int32),
            pltpu.VMEM((_CCH, D), jnp.float32),
            pltpu.VMEM((_CCH, D), jnp.float32),
            pltpu.VMEM((_CCH, D), jnp.float32),
            pltpu.SemaphoreType.DMA,
            pltpu.SemaphoreType.DMA,
        ],
    )
    def k(sh_hbm, yg_hbm, p0_hbm, p1_hbm, out_hbm, p0_v, p1_v,
          a_v, b_v, c_v, sem0, sem1):
        wid = lax.axis_index("s") * 2 + lax.axis_index("c")
        base = wid * tpw
        pltpu.sync_copy(p0_hbm.at[wid], p0_v)
        pltpu.sync_copy(p1_hbm.at[wid], p1_v)

        def body(g, carry):
            cpa = pltpu.async_copy(yg_hbm.at[p0_v.at[g]], a_v, sem0)
            cpb = pltpu.async_copy(yg_hbm.at[p1_v.at[g]], b_v, sem1)
            pltpu.sync_copy(sh_hbm.at[pl.ds(base + g * _CCH, _CCH)], c_v)
            cpa.wait()
            cpb.wait()

            def row(r, carry2):
                def col(kk, carry3):
                    for q in range(4):
                        sl = pl.ds(kk * 64 + q * 16, 16)
                        plsc.addupdate(c_v.at[r, sl], a_v[r, sl] + b_v[r, sl])
                    return carry3
                lax.fori_loop(0, D // 64, col, 0)
                return carry2

            lax.fori_loop(0, _CCH, row, 0)
            pltpu.sync_copy(c_v, out_hbm.at[pl.ds(base + g * _CCH, _CCH)])
            return carry

        lax.fori_loop(0, nit, body, 0)

    return k(out_sh, yg, pos0.reshape(_NW, nit, _CCH), pos1.reshape(_NW, nit, _CCH))


def _routed_body(be_ref, acc_ref, x_ref, wg_ref, wu_ref, wd_ref, ws_ref, out_ref):
    f = pl.program_id(0)
    b = pl.program_id(1)

    @pl.when(be_ref[b] < E)
    def _():
        x = x_ref[...]
        g = jnp.dot(x, wg_ref[0], preferred_element_type=jnp.float32)
        u = jnp.dot(x, wu_ref[0], preferred_element_type=jnp.float32)
        h = (g * jax.nn.sigmoid(g)) * u
        contrib = jnp.dot(h, wd_ref[0], preferred_element_type=jnp.float32)
        res = jnp.where(f == 0, contrib, acc_ref[...] + contrib)
        res = jnp.where(f == NF - 1, res * ws_ref[0, 0, :][:, None], res)
        out_ref[...] = res


def _routed_mlp(block_expert, xg, Wg, Wu, Wd, wslot):
    grid_spec = pltpu.PrefetchScalarGridSpec(
        num_scalar_prefetch=1,
        grid=(NF, NB),
        in_specs=[
            pl.BlockSpec((BM, D), lambda f, b, be: (b, 0)),
            pl.BlockSpec((BM, D), lambda f, b, be: (b, 0)),
            pl.BlockSpec((1, D, BF), lambda f, b, be: (jnp.minimum(be[b], E - 1), 0, f)),
            pl.BlockSpec((1, D, BF), lambda f, b, be: (jnp.minimum(be[b], E - 1), 0, f)),
            pl.BlockSpec((1, BF, D), lambda f, b, be: (jnp.minimum(be[b], E - 1), f, 0)),
            pl.BlockSpec((1, 1, BM), lambda f, b, be: (b, 0, 0)),
        ],
        out_specs=pl.BlockSpec((BM, D), lambda f, b, be: (b, 0)),
    )
    acc = jnp.zeros((P, D), jnp.float32)
    return pl.pallas_call(
        _routed_body,
        grid_spec=grid_spec,
        out_shape=jax.ShapeDtypeStruct((P, D), jnp.float32),
        input_output_aliases={1: 0},
    )(block_expert, acc, xg, Wg, Wu, Wd, wslot.reshape(NB, 1, BM))


def _shared_body(x_ref, wg_ref, wu_ref, wd_ref, out_ref):
    si = pl.program_id(0)
    f = pl.program_id(1)

    @pl.when((si == 0) & (f == 0))
    def _():
        out_ref[...] = jnp.zeros_like(out_ref)

    x = x_ref[...]
    g = jnp.dot(x, wg_ref[0], preferred_element_type=jnp.float32)
    u = jnp.dot(x, wu_ref[0], preferred_element_type=jnp.float32)
    h = (g * jax.nn.sigmoid(g)) * u
    out_ref[...] += jnp.dot(h, wd_ref[0], preferred_element_type=jnp.float32)


BF2 = 256
NF2 = F // BF2


def _shared_mlp(x, Wg_s, Wu_s, Wd_s):
    return pl.pallas_call(
        _shared_body,
        grid=(NSH, NF2),
        in_specs=[
            pl.BlockSpec((T, D), lambda si, f: (0, 0)),
            pl.BlockSpec((1, D, BF2), lambda si, f: (si, 0, f)),
            pl.BlockSpec((1, D, BF2), lambda si, f: (si, 0, f)),
            pl.BlockSpec((1, BF2, D), lambda si, f: (si, f, 0)),
        ],
        out_specs=pl.BlockSpec((T, D), lambda si, f: (0, 0)),
        out_shape=jax.ShapeDtypeStruct((T, D), jnp.float32),
    )(x, Wg_s, Wu_s, Wd_s)


def kernel(hidden_states, gate_w, Wg, Wu, Wd, Wg_s, Wu_s, Wd_s):
    b, s, d = hidden_states.shape
    x = hidden_states.reshape(-1, d)

    i1, i2, w1, w2 = _router(x, gate_w)
    eflat = jnp.stack([i1.reshape(-1), i2.reshape(-1)], axis=-1).reshape(-1)
    wflat = jnp.stack([w1.reshape(-1), w2.reshape(-1)], axis=-1).reshape(-1)

    xg, wslot, pos0, pos1, block_expert = _sc_dispatch_gather(x, eflat, wflat)

    yg = _routed_mlp(block_expert, xg, Wg, Wu, Wd, wslot)
    out_sh = _shared_mlp(x, Wg_s, Wu_s, Wd_s)
    out = _sc_combine(out_sh, yg, pos0, pos1)
    return out.reshape(b, s, d)


# speedup vs baseline: 1.2763x; 1.2697x over previous
import functools

import jax
import jax.numpy as jnp
from jax import lax
from jax.experimental import pallas as pl
from jax.experimental.pallas import tpu as pltpu
from jax.experimental.pallas import tpu_sc as plsc

D, F, E, K, NSH = 2048, 5120, 8, 2, 2
T = 2048
BM = 512
BF = 512
NF = F // BF
NB = 15
P = NB * BM
BM2 = 512

_NW = 32
_GCH = 8
_CCH = 16

def _mesh():
    return plsc.VectorSubcoreMesh(core_axis_name="c", subcore_axis_name="s",
                                  num_cores=2, num_subcores=16)


RBM = 256


def _router_body(x_ref, gw_ref, e0_ref, e1_ref, w0_ref, w1_ref):
    l = jnp.dot(x_ref[...], gw_ref[...], preferred_element_type=jnp.float32)
    sc = jax.nn.sigmoid(l)
    s1 = sc[:, 0]
    i1 = jnp.zeros((RBM,), jnp.int32)
    for e in range(1, E):
        upd = sc[:, e] > s1
        s1 = jnp.where(upd, sc[:, e], s1)
        i1 = jnp.where(upd, e, i1)
    s2 = jnp.full((RBM,), -jnp.inf, jnp.float32)
    i2 = jnp.zeros((RBM,), jnp.int32)
    for e in range(E):
        cand = jnp.logical_and(i1 != e, sc[:, e] > s2)
        s2 = jnp.where(cand, sc[:, e], s2)
        i2 = jnp.where(cand, e, i2)
    tot = s1 + s2
    e0_ref[0, 0, :] = i1
    e1_ref[0, 0, :] = i2
    w0_ref[0, 0, :] = s1 / tot
    w1_ref[0, 0, :] = s2 / tot


def _router(x, gate_w):
    gwp = jnp.zeros((D, 128), jnp.float32).at[:, :E].set(gate_w.T)
    nblk = T // RBM
    outs = pl.pallas_call(
        _router_body,
        grid=(nblk,),
        in_specs=[
            pl.BlockSpec((RBM, D), lambda tb: (tb, 0)),
            pl.BlockSpec((D, 128), lambda tb: (0, 0)),
        ],
        out_specs=[
            pl.BlockSpec((1, 1, RBM), lambda tb: (tb, 0, 0)),
            pl.BlockSpec((1, 1, RBM), lambda tb: (tb, 0, 0)),
            pl.BlockSpec((1, 1, RBM), lambda tb: (tb, 0, 0)),
            pl.BlockSpec((1, 1, RBM), lambda tb: (tb, 0, 0)),
        ],
        out_shape=[
            jax.ShapeDtypeStruct((nblk, 1, RBM), jnp.int32),
            jax.ShapeDtypeStruct((nblk, 1, RBM), jnp.int32),
            jax.ShapeDtypeStruct((nblk, 1, RBM), jnp.float32),
            jax.ShapeDtypeStruct((nblk, 1, RBM), jnp.float32),
        ],
    )(x, gwp)
    return outs


def _sc_dispatch_gather(x, eflat, wflat):
    NCH = (T * K) // 16
    bpw = P // _NW
    nit = bpw // _GCH

    @functools.partial(
        pl.kernel, mesh=_mesh(),
        compiler_params=pltpu.CompilerParams(
            needs_layout_passes=False, use_tc_tiling_on_sc=True),
        out_type=(
            jax.ShapeDtypeStruct((P, D), jnp.float32),
            jax.ShapeDtypeStruct((P,), jnp.float32),
            jax.ShapeDtypeStruct((T,), jnp.int32),
            jax.ShapeDtypeStruct((T,), jnp.int32),
            jax.ShapeDtypeStruct((32,), jnp.int32),
        ),
        scratch_types=[
            pltpu.VMEM((T * K,), jnp.int32),
            pltpu.VMEM((T * K,), jnp.float32),
            pltpu.VMEM((P,), jnp.int32),
            pltpu.VMEM((P,), jnp.float32),
            pltpu.VMEM((T,), jnp.int32),
            pltpu.VMEM((T,), jnp.int32),
            pltpu.VMEM((16,), jnp.int32),
            pltpu.VMEM((32,), jnp.int32),
            pltpu.VMEM((2, _GCH, D), jnp.float32),
            pltpu.SemaphoreType.DMA,
            pltpu.SemaphoreType.DMA,
        ],
    )
    def k(x_hbm, e_hbm, w_hbm, xg_hbm, ws_hbm, p0_hbm, p1_hbm, be_hbm,
          e_v, w_v, perm_v, ws_v, pos0_v, pos1_v, base_ref, be_v, rows_v,
          sem0, sem1):
        wid = lax.axis_index("s") * 2 + lax.axis_index("c")
        base = wid * bpw
        pltpu.sync_copy(e_hbm, e_v)
        pltpu.sync_copy(w_hbm, w_v)
        lanes = lax.iota(jnp.int32, 16)

        def initp(g, carry):
            perm_v[pl.ds(g * 16, 16)] = (g * 16 + lanes) & (T - 1)
            ws_v[pl.ds(g * 16, 16)] = jnp.zeros((16,), jnp.float32)
            return carry

        lax.fori_loop(0, P // 16, initp, 0)

        def hist_body(g, hist):
            ev = e_v[pl.ds(g * 16, 16)]
            for e in range(E):
                c = plsc.all_reduce_population_count(ev == e)
                hist = hist + jnp.where(lanes == e, c, 0)
            return hist

        hist = lax.fori_loop(0, NCH, hist_body, jnp.zeros((16,), jnp.int32))
        padded = ((hist + (BM - 1)) >> 9) << 9
        cum = plsc.cumsum(padded)
        poff = cum - padded
        base_ref[...] = poff

        @pl.when(wid == 0)
        def _():
            for ch in range(2):
                bstart = (ch * 16 + lanes) * BM
                acc = jnp.zeros((16,), jnp.int32)
                for e in range(E):
                    acc = acc + jnp.where(bstart >= cum[e], 1, 0)
                be_v[pl.ds(ch * 16, 16)] = acc
            pltpu.sync_copy(be_v, be_hbm)

        def body(g, carry):
            ev = e_v[pl.ds(g * 16, 16)]
            wv = w_v[pl.ds(g * 16, 16)]
            basev = plsc.load_gather(base_ref, [ev])
            rank = jnp.zeros((16,), jnp.int32)
            newbase = base_ref[...]
            for e in range(E):
                m = ev == e
                cs = plsc.cumsum(jnp.where(m, 1, 0))
                rank = jnp.where(m, cs - 1, rank)
                cnt = plsc.all_reduce_population_count(m)
                newbase = newbase + jnp.where(lanes == e, cnt, 0)
            base_ref[...] = newbase
            dest = basev + rank
            tokv = (g * 16 + lanes) >> 1
            plsc.store_scatter(perm_v, [dest], tokv)
            plsc.store_scatter(ws_v, [dest], wv)
            even = (lanes & 1) == 0
            plsc.store_scatter(pos0_v, [tokv], dest, mask=even)
            plsc.store_scatter(pos1_v, [tokv], dest, mask=jnp.logical_not(even))
            return carry

        lax.fori_loop(0, NCH, body, 0)

        @pl.when(wid == 0)
        def _():
            pltpu.sync_copy(ws_v, ws_hbm)
            pltpu.sync_copy(pos0_v, p0_hbm)
            pltpu.sync_copy(pos1_v, p1_hbm)

        def idx(g):
            return perm_v.at[pl.ds(base + g * _GCH, _GCH)]

        pltpu.async_copy(x_hbm.at[idx(0)], rows_v.at[0], sem0)

        def gbody(h, carry):
            g0 = 2 * h

            @pl.when(g0 + 1 < nit)
            def _():
                pltpu.async_copy(x_hbm.at[idx(g0 + 1)], rows_v.at[1], sem1)

            pltpu.make_async_copy(x_hbm.at[idx(0)], rows_v.at[0], sem0).wait()
            pltpu.sync_copy(rows_v.at[0], xg_hbm.at[pl.ds(base + g0 * _GCH, _GCH)])

            @pl.when(g0 + 2 < nit)
            def _():
                pltpu.async_copy(x_hbm.at[idx(g0 + 2)], rows_v.at[0], sem0)

            @pl.when(g0 + 1 < nit)
            def _():
                pltpu.make_async_copy(x_hbm.at[idx(0)], rows_v.at[1], sem1).wait()
                pltpu.sync_copy(rows_v.at[1],
                                xg_hbm.at[pl.ds(base + (g0 + 1) * _GCH, _GCH)])

            return carry

        lax.fori_loop(0, (nit + 1) // 2, gbody, 0)

    return k(x, eflat, wflat)


def _sc_combine(out_sh, yg, pos0, pos1):
    tpw = T // _NW
    nit = tpw // _CCH

    @functools.partial(
        pl.kernel, mesh=_mesh(),
        compiler_params=pltpu.CompilerParams(use_tc_tiling_on_sc=True),
        out_type=jax.ShapeDtypeStruct((T, D), jnp.float32),
        scratch_types=[
            pltpu.VMEM((nit, _CCH), jnp.int32),
            pltpu.VMEM((nit, _CCH), jnp.int32),
            pltpu.VMEM((_CCH, D), jnp.float32),
            pltpu.VMEM((_CCH, D), jnp.float32),
            pltpu.VMEM((_CCH, D), jnp.float32),
            pltpu.SemaphoreType.DMA,
            pltpu.SemaphoreType.DMA,
        ],
    )
    def k(sh_hbm, yg_hbm, p0_hbm, p1_hbm, out_hbm, p0_v, p1_v,
          a_v, b_v, c_v, sem0, sem1):
        wid = lax.axis_index("s") * 2 + lax.axis_index("c")
        base = wid * tpw
        pltpu.sync_copy(p0_hbm.at[wid], p0_v)
        pltpu.sync_copy(p1_hbm.at[wid], p1_v)

        def body(g, carry):
            cpa = pltpu.async_copy(yg_hbm.at[p0_v.at[g]], a_v, sem0)
            cpb = pltpu.async_copy(yg_hbm.at[p1_v.at[g]], b_v, sem1)
            pltpu.sync_copy(sh_hbm.at[pl.ds(base + g * _CCH, _CCH)], c_v)
            cpa.wait()
            cpb.wait()

            def row(r, carry2):
                def col(kk, carry3):
                    for q in range(4):
                        sl = pl.ds(kk * 64 + q * 16, 16)
                        plsc.addupdate(c_v.at[r, sl], a_v[r, sl] + b_v[r, sl])
                    return carry3
                lax.fori_loop(0, D // 64, col, 0)
                return carry2

            lax.fori_loop(0, _CCH, row, 0)
            pltpu.sync_copy(c_v, out_hbm.at[pl.ds(base + g * _CCH, _CCH)])
            return carry

        lax.fori_loop(0, nit, body, 0)

    return k(out_sh, yg, pos0.reshape(_NW, nit, _CCH), pos1.reshape(_NW, nit, _CCH))


def _routed_body(be_ref, x_ref, wg_ref, wu_ref, wd_ref, ws_ref, out_ref):
    b = pl.program_id(0)
    f = pl.program_id(1)

    @pl.when(f == 0)
    def _():
        out_ref[...] = jnp.zeros_like(out_ref)

    @pl.when(be_ref[b] < E)
    def _():
        x = x_ref[...]
        g = jnp.dot(x, wg_ref[0], preferred_element_type=jnp.float32)
        u = jnp.dot(x, wu_ref[0], preferred_element_type=jnp.float32)
        h = (g * jax.nn.sigmoid(g)) * u
        out_ref[...] += jnp.dot(h, wd_ref[0], preferred_element_type=jnp.float32)

    @pl.when(f == NF - 1)
    def _():
        out_ref[...] *= ws_ref[0, 0, :][:, None]


def _routed_mlp(block_expert, xg, Wg, Wu, Wd, wslot):
    grid_spec = pltpu.PrefetchScalarGridSpec(
        num_scalar_prefetch=1,
        grid=(NB, NF),
        in_specs=[
            pl.BlockSpec((BM, D), lambda b, f, be: (b, 0)),
            pl.BlockSpec((1, D, BF), lambda b, f, be: (jnp.minimum(be[b], E - 1), 0, f)),
            pl.BlockSpec((1, D, BF), lambda b, f, be: (jnp.minimum(be[b], E - 1), 0, f)),
            pl.BlockSpec((1, BF, D), lambda b, f, be: (jnp.minimum(be[b], E - 1), f, 0)),
            pl.BlockSpec((1, 1, BM), lambda b, f, be: (b, 0, 0)),
        ],
        out_specs=pl.BlockSpec((BM, D), lambda b, f, be: (b, 0)),
    )
    return pl.pallas_call(
        _routed_body,
        grid_spec=grid_spec,
        out_shape=jax.ShapeDtypeStruct((P, D), jnp.float32),
    )(block_expert, xg, Wg, Wu, Wd, wslot.reshape(NB, 1, BM))


def _shared_body(x_ref, wg_ref, wu_ref, wd_ref, out_ref):
    si = pl.program_id(0)
    f = pl.program_id(1)

    @pl.when((si == 0) & (f == 0))
    def _():
        out_ref[...] = jnp.zeros_like(out_ref)

    x = x_ref[...]
    g = jnp.dot(x, wg_ref[0], preferred_element_type=jnp.float32)
    u = jnp.dot(x, wu_ref[0], preferred_element_type=jnp.float32)
    h = (g * jax.nn.sigmoid(g)) * u
    out_ref[...] += jnp.dot(h, wd_ref[0], preferred_element_type=jnp.float32)


BF2 = 256
NF2 = F // BF2


def _shared_mlp(x, Wg_s, Wu_s, Wd_s):
    return pl.pallas_call(
        _shared_body,
        grid=(NSH, NF2),
        in_specs=[
            pl.BlockSpec((T, D), lambda si, f: (0, 0)),
            pl.BlockSpec((1, D, BF2), lambda si, f: (si, 0, f)),
            pl.BlockSpec((1, D, BF2), lambda si, f: (si, 0, f)),
            pl.BlockSpec((1, BF2, D), lambda si, f: (si, f, 0)),
        ],
        out_specs=pl.BlockSpec((T, D), lambda si, f: (0, 0)),
        out_shape=jax.ShapeDtypeStruct((T, D), jnp.float32),
    )(x, Wg_s, Wu_s, Wd_s)


def kernel(hidden_states, gate_w, Wg, Wu, Wd, Wg_s, Wu_s, Wd_s):
    b, s, d = hidden_states.shape
    x = hidden_states.reshape(-1, d)

    i1, i2, w1, w2 = _router(x, gate_w)
    eflat = jnp.stack([i1.reshape(-1), i2.reshape(-1)], axis=-1).reshape(-1)
    wflat = jnp.stack([w1.reshape(-1), w2.reshape(-1)], axis=-1).reshape(-1)

    xg, wslot, pos0, pos1, block_expert = _sc_dispatch_gather(x, eflat, wflat)

    yg = _routed_mlp(block_expert, xg, Wg, Wu, Wd, wslot)
    out_sh = _shared_mlp(x, Wg_s, Wu_s, Wd_s)
    out = _sc_combine(out_sh, yg, pos0, pos1)
    return out.reshape(b, s, d)
